# baseline (speedup 1.0000x reference)
import functools

import jax
import jax.numpy as jnp
from jax import lax
from jax.experimental import pallas as pl
from jax.experimental.pallas import tpu as pltpu
from jax.experimental.pallas import tpu_sc as plsc

N = 10000
E = 320000
D_IN = 128
D_EDGE = 16
D_MID = 128
D_OUT = 128

NC = 2
NS = 16
NW = NC * NS
C = 64
NCH = 160
NCH0 = 256
NCH1 = 64
EPT = NCH * C
EP = NW * EPT
NPAD = 10112
DUMMY = NPAD - 1
RPT = NPAD // NS
DPT = NPAD // NS

_sc_mesh = plsc.VectorSubcoreMesh(core_axis_name="c", subcore_axis_name="s")


@functools.partial(
    pl.kernel,
    out_type=jax.ShapeDtypeStruct((NC, NPAD, D_MID), jnp.float32),
    mesh=_sc_mesh,
    scratch_types=[
        [pltpu.VMEM((C,), jnp.int32)] * 2,
        [pltpu.VMEM((C,), jnp.int32)] * 2,
        [pltpu.VMEM((C,), jnp.int32)] * 2,
        [pltpu.VMEM((C, D_MID), jnp.float32)] * 2,
        [pltpu.VMEM((C, D_MID), jnp.float32)] * 2,
        pltpu.VMEM_SHARED((NPAD, D_MID), jnp.float32),
        [pltpu.SemaphoreType.DMA] * 2,
        [pltpu.SemaphoreType.DMA] * 2,
        [pltpu.SemaphoreType.DMA] * 2,
        [pltpu.SemaphoreType.DMA] * 2,
    ],
)
def _sc_edge_kernel(xa, eb, col1d, row1d, zseg,
                    seg_out,
                    colv, rowv, srow, gbuf, ebuf, acc_seg,
                    sem_i, sem_g, sem_e, sem_s):
    c = lax.axis_index("c")
    s = lax.axis_index("s")

    pltpu.sync_copy(zseg, acc_seg.at[pl.ds(s * RPT, RPT)])
    nch = jnp.where(c == 0, NCH0, NCH1)
    ch0 = jnp.where(c == 0, s * NCH0, NS * NCH0 + s * NCH1)
    plsc.subcore_barrier()

    def issue_idx(p, k):
        base = (ch0 + k) * C
        pltpu.async_copy(col1d.at[pl.ds(base, C)], colv[p], sem_i[p])
        pltpu.async_copy(row1d.at[pl.ds(base, C)], rowv[p], sem_i[p])

    def wait_idx(p):
        pltpu.make_async_copy(col1d.at[pl.ds(0, C)], colv[p], sem_i[p]).wait()
        pltpu.make_async_copy(row1d.at[pl.ds(0, C)], rowv[p], sem_i[p]).wait()

    def issue_data(p, k):
        base = (ch0 + k) * C
        pltpu.async_copy(eb.at[pl.ds(base, C)], ebuf[p], sem_e[p])
        pltpu.async_copy(xa.at[colv[p]], gbuf[p], sem_g[p])

    def wait_data(p):
        pltpu.make_async_copy(eb.at[pl.ds(0, C)], ebuf[p], sem_e[p]).wait()
        pltpu.make_async_copy(xa.at[colv[p]], gbuf[p], sem_g[p]).wait()

    def issue_scatter(p):
        pltpu.async_copy(gbuf[p], acc_seg.at[srow[p]], sem_s[p], add=True)

    def wait_scatter(p):
        pltpu.make_async_copy(gbuf[p], acc_seg.at[srow[p]], sem_s[p]).wait()

    def compute(p):
        @plsc.parallel_loop(0, C // 16, unroll=2)
        def _grp(t):
            sl = pl.ds(t * 16, 16)
            srow[p][sl] = rowv[p][sl]

        @plsc.parallel_loop(0, C, unroll=4)
        def _edge(i):
            for j in range(D_MID // 16):
                sl = pl.ds(j * 16, 16)
                gbuf[p][i, sl] = jnp.maximum(
                    gbuf[p][i, sl] + ebuf[p][i, sl], 0.0)

    issue_idx(0, 0)
    issue_idx(1, 1)
    wait_idx(0)
    issue_data(0, 0)
    nch_half = nch // 2

    @pl.loop(0, nch_half)
    def _pipe(g):
        not_last = g < nch_half - 1
        for p in (0, 1):
            m = 2 * g + p
            q = 1 - p

            if p == 0:
                @pl.when(g > 0)
                def _():
                    wait_scatter(q)

                wait_idx(q)
                issue_data(q, m + 1)
            else:
                wait_scatter(q)

                @pl.when(not_last)
                def _():
                    wait_idx(q)
                    issue_data(q, m + 1)

            wait_data(p)
            compute(p)
            issue_scatter(p)

            @pl.when(not_last)
            def _():
                issue_idx(p, m + 2)

    wait_scatter(1)
    plsc.subcore_barrier()
    r0 = s * DPT
    pltpu.sync_copy(acc_seg.at[pl.ds(r0, DPT)], seg_out.at[c, pl.ds(r0, DPT)])


@functools.partial(
    pl.kernel,
    out_type=jax.ShapeDtypeStruct((NC, NPAD, D_MID), jnp.float32),
    mesh=_sc_mesh,
    scratch_types=[
        [pltpu.VMEM((C,), jnp.int32)] * 2,
        [pltpu.VMEM((C,), jnp.int32)] * 2,
        pltpu.VMEM((C, D_MID), jnp.float32),
        pltpu.VMEM_SHARED((NPAD, D_MID), jnp.float32),
        [pltpu.SemaphoreType.DMA] * 2,
        [pltpu.SemaphoreType.DMA] * 2,
    ],
)
def _sc_count_kernel(row1d, zcnt, ones_in, cnt_out,
                     rowv, srow, onesv, acc_cnt, sem_i, sem_s):
    c = lax.axis_index("c")
    s = lax.axis_index("s")
    wid = c * NS + s

    pltpu.sync_copy(zcnt, acc_cnt.at[pl.ds(s * RPT, RPT)])
    pltpu.sync_copy(ones_in, onesv)
    ch0 = wid * NCH
    plsc.subcore_barrier()

    def issue_idx(p, k):
        base = (ch0 + k) * C
        pltpu.async_copy(row1d.at[pl.ds(base, C)], rowv[p], sem_i[p])

    def wait_idx(p):
        pltpu.make_async_copy(row1d.at[pl.ds(0, C)], rowv[p], sem_i[p]).wait()

    def wait_scatter(p):
        pltpu.make_async_copy(onesv, acc_cnt.at[srow[p]], sem_s[p]).wait()

    issue_idx(0, 0)
    issue_idx(1, 1)

    @pl.loop(0, NCH // 2)
    def _pipe(g):
        not_last = g < NCH // 2 - 1
        for p in (0, 1):
            m = 2 * g + p
            wait_idx(p)

            @pl.when(g > 0)
            def _():
                wait_scatter(p)

            @plsc.parallel_loop(0, C // 16, unroll=2)
            def _cp(t):
                sl = pl.ds(t * 16, 16)
                srow[p][sl] = rowv[p][sl]

            pltpu.async_copy(onesv, acc_cnt.at[srow[p]], sem_s[p], add=True)

            @pl.when(not_last)
            def _():
                issue_idx(p, m + 2)

    wait_scatter(0)
    wait_scatter(1)
    plsc.subcore_barrier()
    r0 = s * DPT
    pltpu.sync_copy(acc_cnt.at[pl.ds(r0, DPT)], cnt_out.at[c, pl.ds(r0, DPT)])


def _xa_kernel(x_ref, w_ref, o_ref):
    o_ref[...] = jnp.dot(x_ref[...], w_ref[...],
                         preferred_element_type=jnp.float32)


def _eb_kernel(ea_ref, w_ref, b_ref, o_ref):
    o_ref[...] = jnp.dot(ea_ref[...], w_ref[...],
                         preferred_element_type=jnp.float32) + b_ref[...]


RB = 2000


def _final_kernel(sp_ref, cp_ref, w2, b2, w3, b3, w4, b4, o_ref):
    seg = sp_ref[0] + sp_ref[1]
    cnt = cp_ref[0, :, 0:1] + cp_ref[1, :, 0:1]
    svec = jnp.dot(seg, w2[...], preferred_element_type=jnp.float32)
    svec = svec + cnt * b2[...]
    mean = svec / jnp.maximum(cnt, 1.0)
    h = jnp.maximum(
        jnp.dot(mean, w3[...], preferred_element_type=jnp.float32) + b3[...],
        0.0)
    o_ref[...] = jnp.dot(h, w4[...],
                         preferred_element_type=jnp.float32) + b4[...]


def kernel(x, edge_index, edge_attr, batch, W1, b1, W2, b2, W3, b3, W4, b4):
    del batch
    row = edge_index[0]
    col = edge_index[1]
    npad = EP - E
    col_p = jnp.concatenate([col, jnp.zeros((npad,), jnp.int32)])
    pad_rows = N + jnp.arange(npad, dtype=jnp.int32) % (NPAD - N)
    row_p = jnp.concatenate([row, pad_rows])

    zcnt = jnp.zeros((RPT, D_MID), jnp.float32)
    ones_in = jnp.zeros((C, D_MID), jnp.float32).at[:, 0].set(1.0)
    cnt_p = _sc_count_kernel(row_p, zcnt, ones_in)

    xa = pl.pallas_call(
        _xa_kernel,
        out_shape=jax.ShapeDtypeStruct((N, D_MID), jnp.float32),
    )(x, W1[:D_IN])

    ea_packed = edge_attr.reshape(E // 8, 8 * D_EDGE)
    w_exp = jnp.kron(jnp.eye(8, dtype=jnp.float32), W1[D_IN:])
    b1_tiled = jnp.tile(b1.reshape(1, D_MID), (1, 8))
    ebv = pl.pallas_call(
        _eb_kernel,
        grid=(EP // 8 // 320,),
        in_specs=[
            pl.BlockSpec((320, 8 * D_EDGE),
                         lambda i: (jnp.minimum(i, E // 8 // 320 - 1), 0)),
            pl.BlockSpec((8 * D_EDGE, 8 * D_MID), lambda i: (0, 0)),
            pl.BlockSpec((1, 8 * D_MID), lambda i: (0, 0)),
        ],
        out_specs=pl.BlockSpec((320, 8 * D_MID), lambda i: (i, 0)),
        out_shape=jax.ShapeDtypeStruct((EP // 8, 8 * D_MID), jnp.float32),
    )(ea_packed, w_exp, b1_tiled)
    ebv = ebv.reshape(EP, D_MID)

    zseg = jnp.zeros((RPT, D_MID), jnp.float32)
    seg_p = _sc_edge_kernel(xa, ebv, col_p, row_p, zseg)

    out = pl.pallas_call(
        _final_kernel,
        grid=(N // RB,),
        in_specs=[
            pl.BlockSpec((NC, RB, D_MID), lambda i: (0, i, 0)),
            pl.BlockSpec((NC, RB, D_MID), lambda i: (0, i, 0)),
            pl.BlockSpec((D_MID, D_MID), lambda i: (0, 0)),
            pl.BlockSpec((1, D_MID), lambda i: (0, 0)),
            pl.BlockSpec((D_MID, D_MID), lambda i: (0, 0)),
            pl.BlockSpec((1, D_MID), lambda i: (0, 0)),
            pl.BlockSpec((D_MID, D_OUT), lambda i: (0, 0)),
            pl.BlockSpec((1, D_OUT), lambda i: (0, 0)),
        ],
        out_specs=pl.BlockSpec((RB, D_OUT), lambda i: (i, 0)),
        out_shape=jax.ShapeDtypeStruct((N, D_OUT), jnp.float32),
    )(seg_p, cnt_p, W2, b2.reshape(1, D_MID), W3, b3.reshape(1, D_MID),
      W4, b4.reshape(1, D_OUT))
    return out


# speedup vs baseline: 1.1898x; 1.1898x over previous
import functools

import jax
import jax.numpy as jnp
from jax import lax
from jax.experimental import pallas as pl
from jax.experimental.pallas import tpu as pltpu
from jax.experimental.pallas import tpu_sc as plsc

N = 10000
E = 320000
D_IN = 128
D_EDGE = 16
D_MID = 128
D_OUT = 128

NC = 2
NS = 16
NW = NC * NS
C = 64
NCH = 160
NCH0 = 256
NCH1 = 64
EPT = NCH * C
EP = NW * EPT
NPAD = 10112
DUMMY = NPAD - 1
RPT = NPAD // NS
DPT = NPAD // NS

_sc_mesh = plsc.VectorSubcoreMesh(core_axis_name="c", subcore_axis_name="s")


@functools.partial(
    pl.kernel,
    out_type=jax.ShapeDtypeStruct((NC, NPAD, D_MID), jnp.float32),
    mesh=_sc_mesh,
    scratch_types=[
        [pltpu.VMEM((C,), jnp.int32)] * 2,
        [pltpu.VMEM((C,), jnp.int32)] * 2,
        [pltpu.VMEM((C,), jnp.int32)] * 2,
        [pltpu.VMEM((C, D_MID), jnp.float32)] * 2,
        [pltpu.VMEM((C // 8, 8 * D_MID), jnp.float32)] * 2,
        pltpu.VMEM_SHARED((NPAD, D_MID), jnp.float32),
        [pltpu.SemaphoreType.DMA] * 2,
        [pltpu.SemaphoreType.DMA] * 2,
        [pltpu.SemaphoreType.DMA] * 2,
        [pltpu.SemaphoreType.DMA] * 2,
    ],
)
def _sc_edge_kernel(xa, eb, col1d, row1d, zseg,
                    seg_out,
                    colv, rowv, srow, gbuf, ebuf, acc_seg,
                    sem_i, sem_g, sem_e, sem_s):
    c = lax.axis_index("c")
    s = lax.axis_index("s")

    pltpu.sync_copy(zseg, acc_seg.at[pl.ds(s * RPT, RPT)])
    nch = jnp.where(c == 0, NCH0, NCH1)
    ch0 = jnp.where(c == 0, s * NCH0, NS * NCH0 + s * NCH1)
    plsc.subcore_barrier()

    def issue_idx(p, k):
        base = (ch0 + k) * C
        pltpu.async_copy(col1d.at[pl.ds(base, C)], colv[p], sem_i[p])
        pltpu.async_copy(row1d.at[pl.ds(base, C)], rowv[p], sem_i[p])

    def wait_idx(p):
        pltpu.make_async_copy(col1d.at[pl.ds(0, C)], colv[p], sem_i[p]).wait()
        pltpu.make_async_copy(row1d.at[pl.ds(0, C)], rowv[p], sem_i[p]).wait()

    def issue_data(p, k):
        base8 = (ch0 + k) * (C // 8)
        pltpu.async_copy(eb.at[pl.ds(base8, C // 8)], ebuf[p], sem_e[p])
        pltpu.async_copy(xa.at[colv[p]], gbuf[p], sem_g[p])

    def wait_data(p):
        pltpu.make_async_copy(eb.at[pl.ds(0, C // 8)], ebuf[p], sem_e[p]).wait()
        pltpu.make_async_copy(xa.at[colv[p]], gbuf[p], sem_g[p]).wait()

    def issue_scatter(p):
        pltpu.async_copy(gbuf[p], acc_seg.at[srow[p]], sem_s[p], add=True)

    def wait_scatter(p):
        pltpu.make_async_copy(gbuf[p], acc_seg.at[srow[p]], sem_s[p]).wait()

    def compute(p):
        @plsc.parallel_loop(0, C // 16, unroll=2)
        def _grp(t):
            sl = pl.ds(t * 16, 16)
            srow[p][sl] = rowv[p][sl]

        @plsc.parallel_loop(0, C, unroll=4)
        def _edge(i):
            er = i >> 3
            ec = (i & 7) * D_MID
            for j in range(D_MID // 16):
                sl = pl.ds(j * 16, 16)
                gbuf[p][i, sl] = jnp.maximum(
                    gbuf[p][i, sl] + ebuf[p][er, pl.ds(ec + j * 16, 16)], 0.0)

    issue_idx(0, 0)
    issue_idx(1, 1)
    wait_idx(0)
    issue_data(0, 0)
    nch_half = nch // 2

    @pl.loop(0, nch_half)
    def _pipe(g):
        not_last = g < nch_half - 1
        for p in (0, 1):
            m = 2 * g + p
            q = 1 - p

            if p == 0:
                @pl.when(g > 0)
                def _():
                    wait_scatter(q)

                wait_idx(q)
                issue_data(q, m + 1)
            else:
                wait_scatter(q)

                @pl.when(not_last)
                def _():
                    wait_idx(q)
                    issue_data(q, m + 1)

            wait_data(p)
            compute(p)
            issue_scatter(p)

            @pl.when(not_last)
            def _():
                issue_idx(p, m + 2)

    wait_scatter(1)
    plsc.subcore_barrier()
    r0 = s * DPT
    pltpu.sync_copy(acc_seg.at[pl.ds(r0, DPT)], seg_out.at[c, pl.ds(r0, DPT)])


@functools.partial(
    pl.kernel,
    out_type=jax.ShapeDtypeStruct((NC, NPAD, D_MID), jnp.float32),
    mesh=_sc_mesh,
    scratch_types=[
        [pltpu.VMEM((C,), jnp.int32)] * 2,
        [pltpu.VMEM((C,), jnp.int32)] * 2,
        pltpu.VMEM((C, D_MID), jnp.float32),
        pltpu.VMEM_SHARED((NPAD, D_MID), jnp.float32),
        [pltpu.SemaphoreType.DMA] * 2,
        [pltpu.SemaphoreType.DMA] * 2,
    ],
)
def _sc_count_kernel(row1d, zcnt, ones_in, cnt_out,
                     rowv, srow, onesv, acc_cnt, sem_i, sem_s):
    c = lax.axis_index("c")
    s = lax.axis_index("s")
    wid = c * NS + s

    pltpu.sync_copy(zcnt, acc_cnt.at[pl.ds(s * RPT, RPT)])
    pltpu.sync_copy(ones_in, onesv)
    ch0 = wid * NCH
    plsc.subcore_barrier()

    def issue_idx(p, k):
        base = (ch0 + k) * C
        pltpu.async_copy(row1d.at[pl.ds(base, C)], rowv[p], sem_i[p])

    def wait_idx(p):
        pltpu.make_async_copy(row1d.at[pl.ds(0, C)], rowv[p], sem_i[p]).wait()

    def wait_scatter(p):
        pltpu.make_async_copy(onesv, acc_cnt.at[srow[p]], sem_s[p]).wait()

    issue_idx(0, 0)
    issue_idx(1, 1)

    @pl.loop(0, NCH // 2)
    def _pipe(g):
        not_last = g < NCH // 2 - 1
        for p in (0, 1):
            m = 2 * g + p
            wait_idx(p)

            @pl.when(g > 0)
            def _():
                wait_scatter(p)

            @plsc.parallel_loop(0, C // 16, unroll=2)
            def _cp(t):
                sl = pl.ds(t * 16, 16)
                srow[p][sl] = rowv[p][sl]

            pltpu.async_copy(onesv, acc_cnt.at[srow[p]], sem_s[p], add=True)

            @pl.when(not_last)
            def _():
                issue_idx(p, m + 2)

    wait_scatter(0)
    wait_scatter(1)
    plsc.subcore_barrier()
    r0 = s * DPT
    pltpu.sync_copy(acc_cnt.at[pl.ds(r0, DPT)], cnt_out.at[c, pl.ds(r0, DPT)])


def _xa_kernel(x_ref, w_ref, o_ref):
    o_ref[...] = jnp.dot(x_ref[...], w_ref[...],
                         preferred_element_type=jnp.float32)


def _eb_kernel(ea_ref, w_ref, b_ref, o_ref):
    o_ref[...] = jnp.dot(ea_ref[...], w_ref[...],
                         preferred_element_type=jnp.float32) + b_ref[...]


RB = 2000


def _final_kernel(sp_ref, cp_ref, w2, b2, w3, b3, w4, b4, o_ref):
    seg = sp_ref[0] + sp_ref[1]
    cnt = cp_ref[0, :, 0:1] + cp_ref[1, :, 0:1]
    svec = jnp.dot(seg, w2[...], preferred_element_type=jnp.float32)
    svec = svec + cnt * b2[...]
    mean = svec / jnp.maximum(cnt, 1.0)
    h = jnp.maximum(
        jnp.dot(mean, w3[...], preferred_element_type=jnp.float32) + b3[...],
        0.0)
    o_ref[...] = jnp.dot(h, w4[...],
                         preferred_element_type=jnp.float32) + b4[...]


def kernel(x, edge_index, edge_attr, batch, W1, b1, W2, b2, W3, b3, W4, b4):
    del batch
    row = edge_index[0]
    col = edge_index[1]
    npad = EP - E
    col_p = jnp.concatenate([col, jnp.zeros((npad,), jnp.int32)])
    pad_rows = N + jnp.arange(npad, dtype=jnp.int32) % (NPAD - N)
    row_p = jnp.concatenate([row, pad_rows])

    zcnt = jnp.zeros((RPT, D_MID), jnp.float32)
    ones_in = jnp.zeros((C, D_MID), jnp.float32).at[:, 0].set(1.0)
    cnt_p = _sc_count_kernel(row_p, zcnt, ones_in)

    xa = pl.pallas_call(
        _xa_kernel,
        out_shape=jax.ShapeDtypeStruct((N, D_MID), jnp.float32),
    )(x, W1[:D_IN])

    ea_packed = edge_attr.reshape(E // 8, 8 * D_EDGE)
    w_exp = jnp.kron(jnp.eye(8, dtype=jnp.float32), W1[D_IN:])
    b1_tiled = jnp.tile(b1.reshape(1, D_MID), (1, 8))
    ebv = pl.pallas_call(
        _eb_kernel,
        grid=(EP // 8 // 320,),
        in_specs=[
            pl.BlockSpec((320, 8 * D_EDGE),
                         lambda i: (jnp.minimum(i, E // 8 // 320 - 1), 0)),
            pl.BlockSpec((8 * D_EDGE, 8 * D_MID), lambda i: (0, 0)),
            pl.BlockSpec((1, 8 * D_MID), lambda i: (0, 0)),
        ],
        out_specs=pl.BlockSpec((320, 8 * D_MID), lambda i: (i, 0)),
        out_shape=jax.ShapeDtypeStruct((EP // 8, 8 * D_MID), jnp.float32),
    )(ea_packed, w_exp, b1_tiled)

    zseg = jnp.zeros((RPT, D_MID), jnp.float32)
    seg_p = _sc_edge_kernel(xa, ebv, col_p, row_p, zseg)

    out = pl.pallas_call(
        _final_kernel,
        grid=(N // RB,),
        in_specs=[
            pl.BlockSpec((NC, RB, D_MID), lambda i: (0, i, 0)),
            pl.BlockSpec((NC, RB, D_MID), lambda i: (0, i, 0)),
            pl.BlockSpec((D_MID, D_MID), lambda i: (0, 0)),
            pl.BlockSpec((1, D_MID), lambda i: (0, 0)),
            pl.BlockSpec((D_MID, D_MID), lambda i: (0, 0)),
            pl.BlockSpec((1, D_MID), lambda i: (0, 0)),
            pl.BlockSpec((D_MID, D_OUT), lambda i: (0, 0)),
            pl.BlockSpec((1, D_OUT), lambda i: (0, 0)),
        ],
        out_specs=pl.BlockSpec((RB, D_OUT), lambda i: (i, 0)),
        out_shape=jax.ShapeDtypeStruct((N, D_OUT), jnp.float32),
    )(seg_p, cnt_p, W2, b2.reshape(1, D_MID), W3, b3.reshape(1, D_MID),
      W4, b4.reshape(1, D_OUT))
    return out


# speedup vs baseline: 1.2543x; 1.0542x over previous
import functools

import jax
import jax.numpy as jnp
from jax import lax
from jax.experimental import pallas as pl
from jax.experimental.pallas import tpu as pltpu
from jax.experimental.pallas import tpu_sc as plsc

N = 10000
E = 320000
D_IN = 128
D_EDGE = 16
D_MID = 128
D_OUT = 128

NC = 2
NS = 16
NW = NC * NS
C = 64
NCH = 160
NCH0 = 288
NCH1 = 32
EPT = NCH * C
EP = NW * EPT
NPAD = 10112
DUMMY = NPAD - 1
RPT = NPAD // NS
DPT = NPAD // NS

_sc_mesh = plsc.VectorSubcoreMesh(core_axis_name="c", subcore_axis_name="s")


@functools.partial(
    pl.kernel,
    out_type=jax.ShapeDtypeStruct((NC, NPAD, D_MID), jnp.float32),
    mesh=_sc_mesh,
    scratch_types=[
        [pltpu.VMEM((C,), jnp.int32)] * 2,
        [pltpu.VMEM((C,), jnp.int32)] * 2,
        [pltpu.VMEM((C,), jnp.int32)] * 2,
        [pltpu.VMEM((C, D_MID), jnp.float32)] * 2,
        [pltpu.VMEM((C // 8, 8 * D_MID), jnp.float32)] * 2,
        pltpu.VMEM_SHARED((NPAD, D_MID), jnp.float32),
        [pltpu.SemaphoreType.DMA] * 2,
        [pltpu.SemaphoreType.DMA] * 2,
        [pltpu.SemaphoreType.DMA] * 2,
        [pltpu.SemaphoreType.DMA] * 2,
    ],
)
def _sc_edge_kernel(xa, eb, col1d, row1d, zseg,
                    seg_out,
                    colv, rowv, srow, gbuf, ebuf, acc_seg,
                    sem_i, sem_g, sem_e, sem_s):
    c = lax.axis_index("c")
    s = lax.axis_index("s")

    pltpu.sync_copy(zseg, acc_seg.at[pl.ds(s * RPT, RPT)])
    nch = jnp.where(c == 0, NCH0, NCH1)
    ch0 = jnp.where(c == 0, s * NCH0, NS * NCH0 + s * NCH1)
    plsc.subcore_barrier()

    def issue_idx(p, k):
        base = (ch0 + k) * C
        pltpu.async_copy(col1d.at[pl.ds(base, C)], colv[p], sem_i[p])
        pltpu.async_copy(row1d.at[pl.ds(base, C)], rowv[p], sem_i[p])

    def wait_idx(p):
        pltpu.make_async_copy(col1d.at[pl.ds(0, C)], colv[p], sem_i[p]).wait()
        pltpu.make_async_copy(row1d.at[pl.ds(0, C)], rowv[p], sem_i[p]).wait()

    def issue_data(p, k):
        base8 = (ch0 + k) * (C // 8)
        pltpu.async_copy(eb.at[pl.ds(base8, C // 8)], ebuf[p], sem_e[p])
        pltpu.async_copy(xa.at[colv[p]], gbuf[p], sem_g[p])

    def wait_data(p):
        pltpu.make_async_copy(eb.at[pl.ds(0, C // 8)], ebuf[p], sem_e[p]).wait()
        pltpu.make_async_copy(xa.at[colv[p]], gbuf[p], sem_g[p]).wait()

    def issue_scatter(p):
        pltpu.async_copy(gbuf[p], acc_seg.at[srow[p]], sem_s[p], add=True)

    def wait_scatter(p):
        pltpu.make_async_copy(gbuf[p], acc_seg.at[srow[p]], sem_s[p]).wait()

    def compute(p):
        @plsc.parallel_loop(0, C // 16, unroll=2)
        def _grp(t):
            sl = pl.ds(t * 16, 16)
            srow[p][sl] = rowv[p][sl]

        @plsc.parallel_loop(0, C, unroll=4)
        def _edge(i):
            er = i >> 3
            ec = (i & 7) * D_MID
            for j in range(D_MID // 16):
                sl = pl.ds(j * 16, 16)
                gbuf[p][i, sl] = jnp.maximum(
                    gbuf[p][i, sl] + ebuf[p][er, pl.ds(ec + j * 16, 16)], 0.0)

    issue_idx(0, 0)
    issue_idx(1, 1)
    wait_idx(0)
    issue_data(0, 0)
    nch_half = nch // 2

    @pl.loop(0, nch_half)
    def _pipe(g):
        not_last = g < nch_half - 1
        for p in (0, 1):
            m = 2 * g + p
            q = 1 - p

            if p == 0:
                @pl.when(g > 0)
                def _():
                    wait_scatter(q)

                wait_idx(q)
                issue_data(q, m + 1)
            else:
                wait_scatter(q)

                @pl.when(not_last)
                def _():
                    wait_idx(q)
                    issue_data(q, m + 1)

            wait_data(p)
            compute(p)
            issue_scatter(p)

            @pl.when(not_last)
            def _():
                issue_idx(p, m + 2)

    wait_scatter(1)
    plsc.subcore_barrier()
    r0 = s * DPT
    pltpu.sync_copy(acc_seg.at[pl.ds(r0, DPT)], seg_out.at[c, pl.ds(r0, DPT)])


@functools.partial(
    pl.kernel,
    out_type=jax.ShapeDtypeStruct((NC, NPAD, D_MID), jnp.float32),
    mesh=_sc_mesh,
    scratch_types=[
        [pltpu.VMEM((C,), jnp.int32)] * 2,
        [pltpu.VMEM((C,), jnp.int32)] * 2,
        pltpu.VMEM((C, D_MID), jnp.float32),
        pltpu.VMEM_SHARED((NPAD, D_MID), jnp.float32),
        [pltpu.SemaphoreType.DMA] * 2,
        [pltpu.SemaphoreType.DMA] * 2,
    ],
)
def _sc_count_kernel(row1d, zcnt, ones_in, cnt_out,
                     rowv, srow, onesv, acc_cnt, sem_i, sem_s):
    c = lax.axis_index("c")
    s = lax.axis_index("s")
    wid = c * NS + s

    pltpu.sync_copy(zcnt, acc_cnt.at[pl.ds(s * RPT, RPT)])
    pltpu.sync_copy(ones_in, onesv)
    ch0 = wid * NCH
    plsc.subcore_barrier()

    def issue_idx(p, k):
        base = (ch0 + k) * C
        pltpu.async_copy(row1d.at[pl.ds(base, C)], rowv[p], sem_i[p])

    def wait_idx(p):
        pltpu.make_async_copy(row1d.at[pl.ds(0, C)], rowv[p], sem_i[p]).wait()

    def wait_scatter(p):
        pltpu.make_async_copy(onesv, acc_cnt.at[srow[p]], sem_s[p]).wait()

    issue_idx(0, 0)
    issue_idx(1, 1)

    @pl.loop(0, NCH // 2)
    def _pipe(g):
        not_last = g < NCH // 2 - 1
        for p in (0, 1):
            m = 2 * g + p
            wait_idx(p)

            @pl.when(g > 0)
            def _():
                wait_scatter(p)

            @plsc.parallel_loop(0, C // 16, unroll=2)
            def _cp(t):
                sl = pl.ds(t * 16, 16)
                srow[p][sl] = rowv[p][sl]

            pltpu.async_copy(onesv, acc_cnt.at[srow[p]], sem_s[p], add=True)

            @pl.when(not_last)
            def _():
                issue_idx(p, m + 2)

    wait_scatter(0)
    wait_scatter(1)
    plsc.subcore_barrier()
    r0 = s * DPT
    pltpu.sync_copy(acc_cnt.at[pl.ds(r0, DPT)], cnt_out.at[c, pl.ds(r0, DPT)])


def _xa_kernel(x_ref, w_ref, o_ref):
    o_ref[...] = jnp.dot(x_ref[...], w_ref[...],
                         preferred_element_type=jnp.float32)


def _eb_kernel(ea_ref, w_ref, b_ref, o_ref):
    o_ref[...] = jnp.dot(ea_ref[...], w_ref[...],
                         preferred_element_type=jnp.float32) + b_ref[...]


RB = 2000


def _final_kernel(sp_ref, cp_ref, w2, b2, w3, b3, w4, b4, o_ref):
    seg = sp_ref[0] + sp_ref[1]
    cnt = cp_ref[0, :, 0:1] + cp_ref[1, :, 0:1]
    svec = jnp.dot(seg, w2[...], preferred_element_type=jnp.float32)
    svec = svec + cnt * b2[...]
    mean = svec / jnp.maximum(cnt, 1.0)
    h = jnp.maximum(
        jnp.dot(mean, w3[...], preferred_element_type=jnp.float32) + b3[...],
        0.0)
    o_ref[...] = jnp.dot(h, w4[...],
                         preferred_element_type=jnp.float32) + b4[...]


def kernel(x, edge_index, edge_attr, batch, W1, b1, W2, b2, W3, b3, W4, b4):
    del batch
    row = edge_index[0]
    col = edge_index[1]
    npad = EP - E
    col_p = jnp.concatenate([col, jnp.zeros((npad,), jnp.int32)])
    pad_rows = N + jnp.arange(npad, dtype=jnp.int32) % (NPAD - N)
    row_p = jnp.concatenate([row, pad_rows])

    zcnt = jnp.zeros((RPT, D_MID), jnp.float32)
    ones_in = jnp.zeros((C, D_MID), jnp.float32).at[:, 0].set(1.0)
    cnt_p = _sc_count_kernel(row_p, zcnt, ones_in)

    xa = pl.pallas_call(
        _xa_kernel,
        out_shape=jax.ShapeDtypeStruct((N, D_MID), jnp.float32),
    )(x, W1[:D_IN])

    ea_packed = edge_attr.reshape(E // 8, 8 * D_EDGE)
    w_exp = jnp.kron(jnp.eye(8, dtype=jnp.float32), W1[D_IN:])
    b1_tiled = jnp.tile(b1.reshape(1, D_MID), (1, 8))
    ebv = pl.pallas_call(
        _eb_kernel,
        grid=(EP // 8 // 320,),
        in_specs=[
            pl.BlockSpec((320, 8 * D_EDGE),
                         lambda i: (jnp.minimum(i, E // 8 // 320 - 1), 0)),
            pl.BlockSpec((8 * D_EDGE, 8 * D_MID), lambda i: (0, 0)),
            pl.BlockSpec((1, 8 * D_MID), lambda i: (0, 0)),
        ],
        out_specs=pl.BlockSpec((320, 8 * D_MID), lambda i: (i, 0)),
        out_shape=jax.ShapeDtypeStruct((EP // 8, 8 * D_MID), jnp.float32),
    )(ea_packed, w_exp, b1_tiled)

    zseg = jnp.zeros((RPT, D_MID), jnp.float32)
    seg_p = _sc_edge_kernel(xa, ebv, col_p, row_p, zseg)

    out = pl.pallas_call(
        _final_kernel,
        grid=(N // RB,),
        in_specs=[
            pl.BlockSpec((NC, RB, D_MID), lambda i: (0, i, 0)),
            pl.BlockSpec((NC, RB, D_MID), lambda i: (0, i, 0)),
            pl.BlockSpec((D_MID, D_MID), lambda i: (0, 0)),
            pl.BlockSpec((1, D_MID), lambda i: (0, 0)),
            pl.BlockSpec((D_MID, D_MID), lambda i: (0, 0)),
            pl.BlockSpec((1, D_MID), lambda i: (0, 0)),
            pl.BlockSpec((D_MID, D_OUT), lambda i: (0, 0)),
            pl.BlockSpec((1, D_OUT), lambda i: (0, 0)),
        ],
        out_specs=pl.BlockSpec((RB, D_OUT), lambda i: (i, 0)),
        out_shape=jax.ShapeDtypeStruct((N, D_OUT), jnp.float32),
    )(seg_p, cnt_p, W2, b2.reshape(1, D_MID), W3, b3.reshape(1, D_MID),
      W4, b4.reshape(1, D_OUT))
    return out
